# baseline (speedup 1.0000x reference)
import functools

import jax
import jax.numpy as jnp
from jax import lax
from jax.experimental import pallas as pl
from jax.experimental.pallas import tpu as pltpu
from jax.experimental.pallas import tpu_sc as plsc

_N = 20000
_NPAD = 20480
_Q = 64
_D = 128
_K = 8
_SUP = 160
_NSUP = _N // _SUP
_CAND = 2048
_BIG = 3.0e7
_INF = float("inf")


def _tc_body(data_ref, xm2_ref, ones_ref, out_ref):
    d = data_ref[...]
    dot = lax.dot_general(
        xm2_ref[...], d, (((1,), (1,)), ((), ())),
        precision=lax.Precision.HIGHEST,
        preferred_element_type=jnp.float32)
    rn = lax.dot_general(
        ones_ref[...], d * d, (((1,), (1,)), ((), ())),
        precision=lax.Precision.HIGHEST,
        preferred_element_type=jnp.float32)
    out_ref[...] = dot + rn


def _scores_tc(data, xm2, ones_row):
    blk = 2048
    grid = _NPAD // blk
    return pl.pallas_call(
        _tc_body,
        grid=(grid,),
        in_specs=[
            pl.BlockSpec((blk, _D), lambda i: (i, 0)),
            pl.BlockSpec((_Q, _D), lambda i: (0, 0)),
            pl.BlockSpec((1, _D), lambda i: (0, 0)),
        ],
        out_specs=pl.BlockSpec((_Q, blk), lambda i: (0, i)),
        out_shape=jax.ShapeDtypeStruct((_Q, _NPAD), jnp.float32),
        compiler_params=pltpu.CompilerParams(
            dimension_semantics=("arbitrary",),
        ),
    )(data, xm2, ones_row)


def _lane_iota():
    return lax.iota(jnp.int32, 16)


def _topk_one_query(score_ref, msc_ref, candv_ref, candi_ref):
    inf16 = jnp.full((16,), _INF, jnp.float32)

    def a_body(ci, macc):
        base = ci * _SUP
        vm = score_ref[pl.ds(base, 16)]
        for j in range(1, 10):
            vm = jnp.minimum(vm, score_ref[pl.ds(base + 16 * j, 16)])
        msc_ref[pl.ds(ci * 16, 16)] = vm
        return jnp.minimum(macc, vm)

    macc = lax.fori_loop(0, _NSUP, a_body, inf16)
    t0 = jnp.float32(0)
    for _ in range(_K):
        t0 = jnp.min(macc)
        macc = jnp.where(macc == t0, _INF, macc)

    def b_body(ci, cnt):
        vm = msc_ref[pl.ds(ci * 16, 16)]
        nhit = plsc.all_reduce_population_count(vm <= t0)[0]

        def collect(cnt):
            base = ci * _SUP
            for j in range(10):
                v = score_ref[pl.ds(base + 16 * j, 16)]
                m = v <= t0
                off = jnp.minimum(cnt, _CAND - 32)
                plsc.store_compressed(candv_ref.at[pl.ds(off, 16)], v, mask=m)
                iv = (_lane_iota() + (base + 16 * j)).astype(jnp.float32)
                plsc.store_compressed(candi_ref.at[pl.ds(off, 16)], iv, mask=m)
                cnt = cnt + plsc.all_reduce_population_count(m)[0]
            return cnt

        return lax.cond(nhit > 0, collect, lambda c: c, cnt)

    cnt = lax.fori_loop(0, _NSUP, b_body, jnp.int32(0))
    tail = jnp.minimum(cnt, _CAND - 16)
    candv_ref[pl.ds(tail, 16)] = inf16
    candi_ref[pl.ds(tail, 16)] = jnp.full((16,), _BIG, jnp.float32)
    nc = (cnt + 15) // 16

    picked = []
    for _ in range(_K):
        def m_body(ci, mv):
            return jnp.minimum(mv, candv_ref[pl.ds(ci * 16, 16)])

        m = jnp.min(lax.fori_loop(0, nc, m_body, inf16))

        def p_body(ci, pv):
            v = candv_ref[pl.ds(ci * 16, 16)]
            vi = candi_ref[pl.ds(ci * 16, 16)]
            return jnp.minimum(pv, jnp.where(v == m, vi, _BIG))

        p = jnp.min(lax.fori_loop(0, nc, p_body,
                                  jnp.full((16,), _BIG, jnp.float32)))

        def x_body(ci, _):
            v = candv_ref[pl.ds(ci * 16, 16)]
            vi = candi_ref[pl.ds(ci * 16, 16)]
            candv_ref[pl.ds(ci * 16, 16)] = jnp.where(vi == p, _INF, v)
            return 0

        lax.fori_loop(0, nc, x_body, 0)
        picked.append(p)
    return picked


def _sc_knn(st, label):
    mesh = plsc.VectorSubcoreMesh(core_axis_name="c", subcore_axis_name="s")

    @functools.partial(
        pl.kernel,
        mesh=mesh,
        out_type=jax.ShapeDtypeStruct((32, 16), jnp.int32),
        scratch_types=[
            pltpu.VMEM((_NPAD,), jnp.float32),
            pltpu.VMEM((_NPAD,), jnp.float32),
            pltpu.SemaphoreType.DMA,
            pltpu.SemaphoreType.DMA,
            pltpu.SemaphoreType.DMA,
            pltpu.VMEM((_N,), jnp.int32),
            pltpu.VMEM((_NSUP * 16,), jnp.float32),
            pltpu.VMEM((_CAND,), jnp.float32),
            pltpu.VMEM((_CAND,), jnp.float32),
            pltpu.VMEM((2, 128), jnp.int32),
            pltpu.VMEM((16,), jnp.int32),
        ],
        compiler_params=pltpu.CompilerParams(needs_layout_passes=False),
    )
    def k(st_hbm, lab_hbm, out_hbm, score_v, score2_v, sem0, sem1, seml,
          lab_v, msc_v, candv_v, candi_v, cnt_v, ans_v):
        wid = lax.axis_index("s") * 2 + lax.axis_index("c")
        cp0 = pltpu.make_async_copy(st_hbm.at[2 * wid], score_v, sem0)
        cp0.start()
        cp1 = pltpu.make_async_copy(st_hbm.at[2 * wid + 1], score2_v, sem1)
        cp1.start()
        cpl = pltpu.make_async_copy(lab_hbm, lab_v, seml)
        cpl.start()

        lane = _lane_iota()
        idxvec = jnp.zeros((16,), jnp.int32)
        for qi, (cp, buf) in enumerate(((cp0, score_v), (cp1, score2_v))):
            cp.wait()
            picked = _topk_one_query(buf, msc_v, candv_v, candi_v)
            for r, p in enumerate(picked):
                idxvec = jnp.where(lane == (qi * 8 + r),
                                   p.astype(jnp.int32), idxvec)

        cpl.wait()
        labs = plsc.load_gather(lab_v, [idxvec])

        for row in range(2):
            for ch in range(8):
                cnt_v[row, pl.ds(ch * 16, 16)] = jnp.zeros((16,), jnp.int32)
        sel = (lane >= 8).astype(jnp.int32)
        plsc.addupdate_scatter(cnt_v, [sel, labs],
                               jnp.ones((16,), jnp.int32))

        answers = []
        for qi in range(2):
            best_cnt = jnp.int32(0)
            best_cls = jnp.int32(0)
            for ch in range(7):
                v = cnt_v[qi, pl.ds(ch * 16, 16)]
                cm = jnp.max(v)
                fi = plsc.all_reduce_ffs(v == cm)[0]
                upd = cm > best_cnt
                best_cnt = jnp.where(upd, cm, best_cnt)
                best_cls = jnp.where(upd, ch * 16 + fi, best_cls)
            answers.append(best_cls)

        av = jnp.where(lane == 0, answers[0],
                       jnp.where(lane == 1, answers[1], 0))
        ans_v[...] = av.astype(jnp.int32)
        pltpu.sync_copy(ans_v, out_hbm.at[wid])

    return k(st, label)


def kernel(data, label, x):
    if x.ndim == 1:
        x = x[None, :]
    assert data.shape == (_N, _D) and x.shape == (_Q, _D)
    st = _scores_tc(data, -2.0 * x, jnp.ones((1, _D), jnp.float32))
    out = _sc_knn(st, label)
    return out[:, :2].reshape(_Q, 1)


# speedup vs baseline: 1.0089x; 1.0089x over previous
import functools

import jax
import jax.numpy as jnp
from jax import lax
from jax.experimental import pallas as pl
from jax.experimental.pallas import tpu as pltpu
from jax.experimental.pallas import tpu_sc as plsc

_N = 20000
_NPAD = 20480
_Q = 64
_D = 128
_K = 8
_SUP = 160
_NSUP = _N // _SUP
_CAND = 2048
_BIG = 3.0e7
_INF = float("inf")


def _tc_body(data_ref, xm2_ref, ones_ref, out_ref):
    d = data_ref[...]
    dot = lax.dot_general(
        xm2_ref[...], d, (((1,), (1,)), ((), ())),
        precision=lax.Precision.HIGHEST,
        preferred_element_type=jnp.float32)
    rn = lax.dot_general(
        ones_ref[...], d * d, (((1,), (1,)), ((), ())),
        precision=lax.Precision.HIGHEST,
        preferred_element_type=jnp.float32)
    out_ref[...] = dot + rn


def _scores_tc(data, xm2, ones_row):
    blk = 2048
    grid = _NPAD // blk
    return pl.pallas_call(
        _tc_body,
        grid=(grid,),
        in_specs=[
            pl.BlockSpec((blk, _D), lambda i: (i, 0)),
            pl.BlockSpec((_Q, _D), lambda i: (0, 0)),
            pl.BlockSpec((1, _D), lambda i: (0, 0)),
        ],
        out_specs=pl.BlockSpec((_Q, blk), lambda i: (0, i)),
        out_shape=jax.ShapeDtypeStruct((_Q, _NPAD), jnp.float32),
        compiler_params=pltpu.CompilerParams(
            dimension_semantics=("arbitrary",),
        ),
    )(data, xm2, ones_row)


def _lane_iota():
    return lax.iota(jnp.int32, 16)


def _topk_one_query(score_ref, msc_ref, candv_ref, candi_ref):
    inf16 = jnp.full((16,), _INF, jnp.float32)

    def a_body(gi, macc):
        for u in range(5):
            ci = gi * 5 + u
            base = ci * _SUP
            vm = score_ref[pl.ds(base, 16)]
            for j in range(1, 10):
                vm = jnp.minimum(vm, score_ref[pl.ds(base + 16 * j, 16)])
            msc_ref[pl.ds(ci * 16, 16)] = vm
            macc = jnp.minimum(macc, vm)
        return macc

    macc = lax.fori_loop(0, _NSUP // 5, a_body, inf16)
    t0 = jnp.float32(0)
    for _ in range(_K):
        t0 = jnp.min(macc)
        macc = jnp.where(macc == t0, _INF, macc)

    def b_group(gi, cnt):
        gmin = msc_ref[pl.ds(gi * 80, 16)]
        for u in range(1, 5):
            gmin = jnp.minimum(gmin, msc_ref[pl.ds(gi * 80 + u * 16, 16)])
        ghit = plsc.all_reduce_population_count(gmin <= t0)[0]

        def scan_group(cnt):
            for u in range(5):
                ci = gi * 5 + u
                vm = msc_ref[pl.ds(ci * 16, 16)]
                nhit = plsc.all_reduce_population_count(vm <= t0)[0]

                def collect(cnt, ci=ci):
                    base = ci * _SUP
                    for j in range(10):
                        v = score_ref[pl.ds(base + 16 * j, 16)]
                        m = v <= t0
                        off = jnp.minimum(cnt, _CAND - 32)
                        plsc.store_compressed(
                            candv_ref.at[pl.ds(off, 16)], v, mask=m)
                        iv = (_lane_iota()
                              + (base + 16 * j)).astype(jnp.float32)
                        plsc.store_compressed(
                            candi_ref.at[pl.ds(off, 16)], iv, mask=m)
                        cnt = cnt + plsc.all_reduce_population_count(m)[0]
                    return cnt

                cnt = lax.cond(nhit > 0, collect, lambda c: c, cnt)
            return cnt

        return lax.cond(ghit > 0, scan_group, lambda c: c, cnt)

    cnt = lax.fori_loop(0, _NSUP // 5, b_group, jnp.int32(0))
    tail = jnp.minimum(cnt, _CAND - 16)
    candv_ref[pl.ds(tail, 16)] = inf16
    candi_ref[pl.ds(tail, 16)] = jnp.full((16,), _BIG, jnp.float32)
    nc = (cnt + 15) // 16

    picked = []
    for _ in range(_K):
        def m_body(ci, mv):
            return jnp.minimum(mv, candv_ref[pl.ds(ci * 16, 16)])

        m = jnp.min(lax.fori_loop(0, nc, m_body, inf16))

        def p_body(ci, pv):
            v = candv_ref[pl.ds(ci * 16, 16)]
            vi = candi_ref[pl.ds(ci * 16, 16)]
            return jnp.minimum(pv, jnp.where(v == m, vi, _BIG))

        p = jnp.min(lax.fori_loop(0, nc, p_body,
                                  jnp.full((16,), _BIG, jnp.float32)))

        def x_body(ci, _):
            v = candv_ref[pl.ds(ci * 16, 16)]
            vi = candi_ref[pl.ds(ci * 16, 16)]
            candv_ref[pl.ds(ci * 16, 16)] = jnp.where(vi == p, _INF, v)
            return 0

        lax.fori_loop(0, nc, x_body, 0)
        picked.append(p)
    return picked


def _sc_knn(st, label):
    mesh = plsc.VectorSubcoreMesh(core_axis_name="c", subcore_axis_name="s")

    @functools.partial(
        pl.kernel,
        mesh=mesh,
        out_type=jax.ShapeDtypeStruct((32, 16), jnp.int32),
        scratch_types=[
            pltpu.VMEM((_NPAD,), jnp.float32),
            pltpu.VMEM((_NPAD,), jnp.float32),
            pltpu.SemaphoreType.DMA,
            pltpu.SemaphoreType.DMA,
            pltpu.SemaphoreType.DMA,
            pltpu.VMEM((_N,), jnp.int32),
            pltpu.VMEM((_NSUP * 16,), jnp.float32),
            pltpu.VMEM((_CAND,), jnp.float32),
            pltpu.VMEM((_CAND,), jnp.float32),
            pltpu.VMEM((2, 128), jnp.int32),
            pltpu.VMEM((16,), jnp.int32),
        ],
        compiler_params=pltpu.CompilerParams(needs_layout_passes=False),
    )
    def k(st_hbm, lab_hbm, out_hbm, score_v, score2_v, sem0, sem1, seml,
          lab_v, msc_v, candv_v, candi_v, cnt_v, ans_v):
        wid = lax.axis_index("s") * 2 + lax.axis_index("c")
        cp0 = pltpu.make_async_copy(st_hbm.at[2 * wid], score_v, sem0)
        cp0.start()
        cp1 = pltpu.make_async_copy(st_hbm.at[2 * wid + 1], score2_v, sem1)
        cp1.start()
        cpl = pltpu.make_async_copy(lab_hbm, lab_v, seml)
        cpl.start()

        lane = _lane_iota()
        idxvec = jnp.zeros((16,), jnp.int32)
        for qi, (cp, buf) in enumerate(((cp0, score_v), (cp1, score2_v))):
            cp.wait()
            picked = _topk_one_query(buf, msc_v, candv_v, candi_v)
            for r, p in enumerate(picked):
                idxvec = jnp.where(lane == (qi * 8 + r),
                                   p.astype(jnp.int32), idxvec)

        cpl.wait()
        labs = plsc.load_gather(lab_v, [idxvec])

        for row in range(2):
            for ch in range(8):
                cnt_v[row, pl.ds(ch * 16, 16)] = jnp.zeros((16,), jnp.int32)
        sel = (lane >= 8).astype(jnp.int32)
        plsc.addupdate_scatter(cnt_v, [sel, labs],
                               jnp.ones((16,), jnp.int32))

        answers = []
        for qi in range(2):
            best_cnt = jnp.int32(0)
            best_cls = jnp.int32(0)
            for ch in range(7):
                v = cnt_v[qi, pl.ds(ch * 16, 16)]
                cm = jnp.max(v)
                fi = plsc.all_reduce_ffs(v == cm)[0]
                upd = cm > best_cnt
                best_cnt = jnp.where(upd, cm, best_cnt)
                best_cls = jnp.where(upd, ch * 16 + fi, best_cls)
            answers.append(best_cls)

        av = jnp.where(lane == 0, answers[0],
                       jnp.where(lane == 1, answers[1], 0))
        ans_v[...] = av.astype(jnp.int32)
        pltpu.sync_copy(ans_v, out_hbm.at[wid])

    return k(st, label)


def kernel(data, label, x):
    if x.ndim == 1:
        x = x[None, :]
    assert data.shape == (_N, _D) and x.shape == (_Q, _D)
    st = _scores_tc(data, -2.0 * x, jnp.ones((1, _D), jnp.float32))
    out = _sc_knn(st, label)
    return out[:, :2].reshape(_Q, 1)


# speedup vs baseline: 1.0139x; 1.0050x over previous
import functools

import jax
import jax.numpy as jnp
from jax import lax
from jax.experimental import pallas as pl
from jax.experimental.pallas import tpu as pltpu
from jax.experimental.pallas import tpu_sc as plsc

_N = 20000
_NPAD = 20480
_Q = 64
_D = 128
_K = 8
_SUP = 160
_NSUP = _N // _SUP
_CAND = 2048
_BIG = 3.0e7
_INF = float("inf")


def _tc_body(data_ref, xm2_ref, ones_ref, out_ref):
    d = data_ref[...]
    dot = lax.dot_general(
        xm2_ref[...], d, (((1,), (1,)), ((), ())),
        precision=lax.Precision.HIGHEST,
        preferred_element_type=jnp.float32)
    rn = lax.dot_general(
        ones_ref[...], d * d, (((1,), (1,)), ((), ())),
        precision=lax.Precision.HIGHEST,
        preferred_element_type=jnp.float32)
    out_ref[...] = dot + rn


def _scores_tc(data, xm2, ones_row):
    blk = 4096
    grid = _NPAD // blk
    return pl.pallas_call(
        _tc_body,
        grid=(grid,),
        in_specs=[
            pl.BlockSpec((blk, _D), lambda i: (i, 0)),
            pl.BlockSpec((_Q, _D), lambda i: (0, 0)),
            pl.BlockSpec((1, _D), lambda i: (0, 0)),
        ],
        out_specs=pl.BlockSpec((_Q, blk), lambda i: (0, i)),
        out_shape=jax.ShapeDtypeStruct((_Q, _NPAD), jnp.float32),
        compiler_params=pltpu.CompilerParams(
            dimension_semantics=("arbitrary",),
        ),
    )(data, xm2, ones_row)


def _lane_iota():
    return lax.iota(jnp.int32, 16)


def _topk_one_query(score_ref, msc_ref, candv_ref, candi_ref):
    inf16 = jnp.full((16,), _INF, jnp.float32)

    def a_body(gi, macc):
        for u in range(5):
            ci = gi * 5 + u
            base = ci * _SUP
            vm = score_ref[pl.ds(base, 16)]
            for j in range(1, 10):
                vm = jnp.minimum(vm, score_ref[pl.ds(base + 16 * j, 16)])
            msc_ref[pl.ds(ci * 16, 16)] = vm
            macc = jnp.minimum(macc, vm)
        return macc

    macc = lax.fori_loop(0, _NSUP // 5, a_body, inf16)
    t0 = jnp.float32(0)
    for _ in range(_K):
        t0 = jnp.min(macc)
        macc = jnp.where(macc == t0, _INF, macc)

    def b_group(gi, cnt):
        gmin = msc_ref[pl.ds(gi * 80, 16)]
        for u in range(1, 5):
            gmin = jnp.minimum(gmin, msc_ref[pl.ds(gi * 80 + u * 16, 16)])
        ghit = plsc.all_reduce_population_count(gmin <= t0)[0]

        def scan_group(cnt):
            for u in range(5):
                ci = gi * 5 + u
                vm = msc_ref[pl.ds(ci * 16, 16)]
                nhit = plsc.all_reduce_population_count(vm <= t0)[0]

                def collect(cnt, ci=ci):
                    base = ci * _SUP
                    for j in range(10):
                        v = score_ref[pl.ds(base + 16 * j, 16)]
                        m = v <= t0
                        off = jnp.minimum(cnt, _CAND - 32)
                        plsc.store_compressed(
                            candv_ref.at[pl.ds(off, 16)], v, mask=m)
                        iv = (_lane_iota()
                              + (base + 16 * j)).astype(jnp.float32)
                        plsc.store_compressed(
                            candi_ref.at[pl.ds(off, 16)], iv, mask=m)
                        cnt = cnt + plsc.all_reduce_population_count(m)[0]
                    return cnt

                cnt = lax.cond(nhit > 0, collect, lambda c: c, cnt)
            return cnt

        return lax.cond(ghit > 0, scan_group, lambda c: c, cnt)

    cnt = lax.fori_loop(0, _NSUP // 5, b_group, jnp.int32(0))
    tail = jnp.minimum(cnt, _CAND - 16)
    candv_ref[pl.ds(tail, 16)] = inf16
    candi_ref[pl.ds(tail, 16)] = jnp.full((16,), _BIG, jnp.float32)
    nc = (cnt + 15) // 16

    picked = []
    for _ in range(_K):
        def m_body(ci, mv):
            return jnp.minimum(mv, candv_ref[pl.ds(ci * 16, 16)])

        m = jnp.min(lax.fori_loop(0, nc, m_body, inf16))

        def p_body(ci, pv):
            v = candv_ref[pl.ds(ci * 16, 16)]
            vi = candi_ref[pl.ds(ci * 16, 16)]
            return jnp.minimum(pv, jnp.where(v == m, vi, _BIG))

        p = jnp.min(lax.fori_loop(0, nc, p_body,
                                  jnp.full((16,), _BIG, jnp.float32)))

        def x_body(ci, _):
            v = candv_ref[pl.ds(ci * 16, 16)]
            vi = candi_ref[pl.ds(ci * 16, 16)]
            candv_ref[pl.ds(ci * 16, 16)] = jnp.where(vi == p, _INF, v)
            return 0

        lax.fori_loop(0, nc, x_body, 0)
        picked.append(p)
    return picked


def _sc_knn(st, label):
    mesh = plsc.VectorSubcoreMesh(core_axis_name="c", subcore_axis_name="s")

    @functools.partial(
        pl.kernel,
        mesh=mesh,
        out_type=jax.ShapeDtypeStruct((32, 16), jnp.int32),
        scratch_types=[
            pltpu.VMEM((_NPAD,), jnp.float32),
            pltpu.VMEM((_NPAD,), jnp.float32),
            pltpu.SemaphoreType.DMA,
            pltpu.SemaphoreType.DMA,
            pltpu.SemaphoreType.DMA,
            pltpu.VMEM((_N,), jnp.int32),
            pltpu.VMEM((_NSUP * 16,), jnp.float32),
            pltpu.VMEM((_CAND,), jnp.float32),
            pltpu.VMEM((_CAND,), jnp.float32),
            pltpu.VMEM((2, 128), jnp.int32),
            pltpu.VMEM((16,), jnp.int32),
        ],
        compiler_params=pltpu.CompilerParams(needs_layout_passes=False),
    )
    def k(st_hbm, lab_hbm, out_hbm, score_v, score2_v, sem0, sem1, seml,
          lab_v, msc_v, candv_v, candi_v, cnt_v, ans_v):
        wid = lax.axis_index("s") * 2 + lax.axis_index("c")
        cp0 = pltpu.make_async_copy(st_hbm.at[2 * wid], score_v, sem0)
        cp0.start()
        cp1 = pltpu.make_async_copy(st_hbm.at[2 * wid + 1], score2_v, sem1)
        cp1.start()
        cpl = pltpu.make_async_copy(lab_hbm, lab_v, seml)
        cpl.start()

        lane = _lane_iota()
        idxvec = jnp.zeros((16,), jnp.int32)
        for qi, (cp, buf) in enumerate(((cp0, score_v), (cp1, score2_v))):
            cp.wait()
            picked = _topk_one_query(buf, msc_v, candv_v, candi_v)
            for r, p in enumerate(picked):
                idxvec = jnp.where(lane == (qi * 8 + r),
                                   p.astype(jnp.int32), idxvec)

        cpl.wait()
        labs = plsc.load_gather(lab_v, [idxvec])

        for row in range(2):
            for ch in range(8):
                cnt_v[row, pl.ds(ch * 16, 16)] = jnp.zeros((16,), jnp.int32)
        sel = (lane >= 8).astype(jnp.int32)
        plsc.addupdate_scatter(cnt_v, [sel, labs],
                               jnp.ones((16,), jnp.int32))

        answers = []
        for qi in range(2):
            best_cnt = jnp.int32(0)
            best_cls = jnp.int32(0)
            for ch in range(7):
                v = cnt_v[qi, pl.ds(ch * 16, 16)]
                cm = jnp.max(v)
                fi = plsc.all_reduce_ffs(v == cm)[0]
                upd = cm > best_cnt
                best_cnt = jnp.where(upd, cm, best_cnt)
                best_cls = jnp.where(upd, ch * 16 + fi, best_cls)
            answers.append(best_cls)

        av = jnp.where(lane == 0, answers[0],
                       jnp.where(lane == 1, answers[1], 0))
        ans_v[...] = av.astype(jnp.int32)
        pltpu.sync_copy(ans_v, out_hbm.at[wid])

    return k(st, label)


def kernel(data, label, x):
    if x.ndim == 1:
        x = x[None, :]
    assert data.shape == (_N, _D) and x.shape == (_Q, _D)
    st = _scores_tc(data, -2.0 * x, jnp.ones((1, _D), jnp.float32))
    out = _sc_knn(st, label)
    return out[:, :2].reshape(_Q, 1)
